# baseline (speedup 1.0000x reference)
import functools

import jax
import jax.numpy as jnp
import numpy as np
from jax import lax
from jax.experimental import pallas as pl
from jax.experimental.pallas import tpu as pltpu
from jax.experimental.pallas import tpu_sc as plsc

_INTERPRET = False

NP = 8192
K = 128
DK = 22
R = 128
NEG = -1e30
POS = 1e30


def _stage2_body(vals_ref, xq_ref, yq_ref, cx_ref, cy_ref,
                 r0_ref, r1_ref, r2_ref,
                 wc_ref, wx_ref, wy_ref, b1_ref,
                 feats_ref, zmax_ref, zmin_ref, s1_ref, s2_ref):
    corr = vals_ref[...]
    dxv = xq_ref[...] - cx_ref[...]
    dyv = yq_ref[...] - cy_ref[...]
    rs = (r0_ref[0, 0], r1_ref[0, 0], r2_ref[0, 0])

    cols = []
    for lvl in range(3):
        r = rs[lvl]
        dv0 = jnp.round(dxv / r)
        dv1 = jnp.round(dyv / r)
        valid = (jnp.abs(dv0) <= 1.0) & (jnp.abs(dv1) <= 1.0)
        cube = (dv0 + 1.0) * 3.0 + (dv1 + 1.0)
        for b in range(9):
            m = valid & (cube == float(b))
            add = jnp.sum(jnp.where(m, corr, 0.0), axis=1)
            cnt = jnp.sum(jnp.where(m, 1.0, 0.0), axis=1)
            cols.append(add / jnp.maximum(cnt, 1.0))
    feats27 = jnp.stack(cols, axis=1)
    feats_ref[...] = jnp.concatenate(
        [feats27, jnp.zeros((R, 5), jnp.float32)], axis=1)

    dist = dxv * dxv + dyv * dyv
    dl = dist[:, :, None]
    dj = dist[:, None, :]
    il = jax.lax.broadcasted_iota(jnp.int32, (K, K), 0)
    ij = jax.lax.broadcasted_iota(jnp.int32, (K, K), 1)
    before = (dl < dj) | ((dl == dj) & (il < ij)[None])
    rank = jnp.sum(before.astype(jnp.float32), axis=1)
    sel = (rank < float(DK))[None]

    wc = wc_ref[...][:, :, None]
    wx = wx_ref[...][:, :, None]
    wy = wy_ref[...][:, :, None]
    b1 = b1_ref[...][:, :, None]
    z = wc * corr[None] + wx * dxv[None] + wy * dyv[None] + b1
    zmax_ref[...] = jnp.max(jnp.where(sel, z, NEG), axis=2)
    zmin_ref[...] = jnp.min(jnp.where(sel, z, POS), axis=2)
    zs = jnp.where(sel, z, 0.0)
    s1_ref[...] = jnp.sum(zs, axis=2)
    s2_ref[...] = jnp.sum(zs * zs, axis=2)


def _stage3_body(feats_ref, zmax_ref, zmin_ref, s1_ref, s2_ref,
                 w1_ref, b1_ref, gnw_ref, gnb_ref, ap_ref, w2_ref, b2_ref,
                 kgnw_ref, kgnb_ref, kap_ref, kw2_ref, kb2_ref, out_ref):
    ic = jax.lax.broadcasted_iota(jnp.int32, (64, 8), 0)
    ig = jax.lax.broadcasted_iota(jnp.int32, (64, 8), 1)
    oneh = ((ic // 8) == ig).astype(jnp.float32)

    x = feats_ref[...].T
    y1 = jnp.dot(w1_ref[...], x, preferred_element_type=jnp.float32)
    y1 = y1 + b1_ref[...]
    nv = jnp.float32(8 * NP)
    sy = jnp.sum(y1, axis=1, keepdims=True)
    mgv = jnp.dot(oneh.T, sy, preferred_element_type=jnp.float32) / nv
    m64v = jnp.dot(oneh, mgv, preferred_element_type=jnp.float32)
    yc = y1 - m64v
    sv = jnp.sum(yc * yc, axis=1, keepdims=True)
    vgv = jnp.dot(oneh.T, sv, preferred_element_type=jnp.float32) / nv
    invv = jnp.dot(oneh, 1.0 / jnp.sqrt(vgv + 1e-5),
                   preferred_element_type=jnp.float32)
    yn = yc * invv * gnw_ref[...] + gnb_ref[...]
    a = ap_ref[0, 0]
    ya = jnp.where(yn >= 0, yn, a * yn)
    voxel = jnp.dot(w2_ref[...], ya, preferred_element_type=jnp.float32)
    voxel = voxel + b2_ref[...]

    n = jnp.float32(8 * NP * DK)
    s1c = jnp.sum(s1_ref[...], axis=1, keepdims=True)
    s2c = jnp.sum(s2_ref[...], axis=1, keepdims=True)
    mg = jnp.dot(oneh.T, s1c, preferred_element_type=jnp.float32) / n
    eg = jnp.dot(oneh.T, s2c, preferred_element_type=jnp.float32) / n
    vg = jnp.maximum(eg - mg * mg, 0.0)
    m64 = jnp.dot(oneh, mg, preferred_element_type=jnp.float32)
    inv64 = jnp.dot(oneh, 1.0 / jnp.sqrt(vg + 1e-5),
                    preferred_element_type=jnp.float32)
    kgnw = kgnw_ref[...]
    znmax = (zmax_ref[...] - m64) * inv64 * kgnw + kgnb_ref[...]
    znmin = (zmin_ref[...] - m64) * inv64 * kgnw + kgnb_ref[...]
    tsel = jnp.where(kgnw >= 0, znmax, znmin)
    ka = kap_ref[0, 0]
    t = jnp.where(tsel >= 0, tsel, ka * tsel)
    knn = jnp.dot(kw2_ref[...], t, preferred_element_type=jnp.float32)
    knn = knn + kb2_ref[...]
    out_ref[...] = voxel + knn


def _run_stage2(vals, xq, yq, coords, r012, knn_w1, knn_b1):
    grid = NP // R
    row_spec = pl.BlockSpec((R, K), lambda i: (i, 0))
    coord_spec = pl.BlockSpec((R, 1), lambda i: (i, 0))
    scalar_spec = pl.BlockSpec((1, 1), lambda i: (0, 0))
    col_spec = pl.BlockSpec((64, 1), lambda i: (0, 0))
    out_specs = [
        pl.BlockSpec((R, 32), lambda i: (i, 0)),
        pl.BlockSpec((64, R), lambda i: (0, i)),
        pl.BlockSpec((64, R), lambda i: (0, i)),
        pl.BlockSpec((64, R), lambda i: (0, i)),
        pl.BlockSpec((64, R), lambda i: (0, i)),
    ]
    out_shape = [
        jax.ShapeDtypeStruct((NP, 32), jnp.float32),
        jax.ShapeDtypeStruct((64, NP), jnp.float32),
        jax.ShapeDtypeStruct((64, NP), jnp.float32),
        jax.ShapeDtypeStruct((64, NP), jnp.float32),
        jax.ShapeDtypeStruct((64, NP), jnp.float32),
    ]
    wc = knn_w1[:, 0:1]
    wx = knn_w1[:, 1:2]
    wy = knn_w1[:, 2:3]
    return pl.pallas_call(
        _stage2_body,
        grid=(grid,),
        in_specs=[row_spec, row_spec, row_spec, coord_spec, coord_spec,
                  scalar_spec, scalar_spec, scalar_spec,
                  col_spec, col_spec, col_spec, col_spec],
        out_specs=out_specs,
        out_shape=out_shape,
        interpret=_INTERPRET,
    )(vals, xq, yq, coords[0, :, 0:1], coords[0, :, 1:2],
      r012[0], r012[1], r012[2], wc, wx, wy, knn_b1[:, None])


def _run_stage3(feats, zmax, zmin, s1, s2, out_w1, out_b1, out_gn_w,
                out_gn_b, out_prelu, out_w2, out_b2, knn_gn_w, knn_gn_b,
                knn_prelu, knn_w2, knn_b2):
    w1p = jnp.concatenate(
        [out_w1, jnp.zeros((64, 5), jnp.float32)], axis=1)
    return pl.pallas_call(
        _stage3_body,
        out_shape=jax.ShapeDtypeStruct((64, NP), jnp.float32),
        interpret=_INTERPRET,
    )(feats, zmax, zmin, s1, s2,
      w1p, out_b1[:, None], out_gn_w[:, None], out_gn_b[:, None],
      out_prelu[:, None], out_w2, out_b2[:, None],
      knn_gn_w[:, None], knn_gn_b[:, None], knn_prelu[:, None],
      knn_w2, knn_b2[:, None])


NW = 32
ROWS_W = NP // NW
MININT = -2147483648


def _mono(v):
    b = plsc.bitcast(v, jnp.int32)
    m = (b >> 31) | jnp.int32(MININT)
    return b ^ m


def _unmono(u):
    t = u >> 31
    return plsc.bitcast(u ^ ((~t) | jnp.int32(MININT)), jnp.float32)


def _digit(u, shift, bits):
    return lax.shift_right_logical(u, shift) & ((1 << bits) - 1)


def _popcnt(mask):
    return jnp.sum(mask.astype(jnp.int32))


def _sc_topk_body(tr_hbm, xyx_hbm, xyy_hbm, out_hbm,
                  rbuf, ua, ia, hist, accu, acci, xx, yy,
                  stage0, stage1, sem_r0, sem_r1, sem_xy):
    wid = lax.axis_index("s") * 2 + lax.axis_index("c")
    lane = lax.iota(jnp.int32, 16)
    ones_i = jnp.ones((16,), jnp.int32)

    pltpu.async_copy(xyx_hbm, xx, sem_xy).wait()
    pltpu.async_copy(xyy_hbm, yy, sem_xy).wait()

    row0 = wid * ROWS_W

    zero_v = jnp.zeros((16,), jnp.int32)

    def zero_hist():
        for r in range(16):
            hist[r, pl.ds(0, 16)] = zero_v
            hist[r, pl.ds(16, 16)] = zero_v

    def scan_hist(krem):
        tot_lo = hist[0, pl.ds(0, 16)]
        tot_hi = hist[0, pl.ds(16, 16)]
        for r in range(1, 16):
            tot_lo = tot_lo + hist[r, pl.ds(0, 16)]
            tot_hi = tot_hi + hist[r, pl.ds(16, 16)]
        s_hi = jnp.sum(tot_hi)
        cum_hi = plsc.cumsum(lax.rev(tot_hi, (0,)))
        cum_lo = plsc.cumsum(lax.rev(tot_lo, (0,)))
        p_hi = plsc.all_reduce_ffs(cum_hi >= krem)
        p_lo = plsc.all_reduce_ffs(cum_lo >= (krem - s_hi))
        dstar_v = jnp.where(s_hi >= krem, 31 - p_hi, 15 - p_lo)
        dh = lane + 16
        csum = (jnp.sum(jnp.where(dh > dstar_v, tot_hi, 0))
                + jnp.sum(jnp.where(lane > dstar_v, tot_lo, 0)))
        cstar = jnp.sum(jnp.where(dh == dstar_v, tot_hi, 0)
                        + jnp.where(lane == dstar_v, tot_lo, 0))
        return csum, dstar_v, cstar

    def append(buf_u, buf_i, base_v, u, idx, mask):
        pos = base_v + plsc.cumsum(mask.astype(jnp.int32)) - 1
        plsc.store_scatter(buf_u, [pos], u, mask=mask)
        plsc.store_scatter(buf_i, [pos], idx, mask=mask)
        return base_v + plsc.all_reduce_population_count(mask)

    def select_row(par):
        zero_hist()

        def h1(i, _):
            u = _mono(rbuf[par, pl.ds(i * 16, 16)])
            d = _digit(u, 27, 5)
            plsc.addupdate_scatter(hist, [lane, d], ones_i)
            return 0

        lax.fori_loop(0, NP // 16, h1, 0)
        csum, dstar_v, cstar = scan_hist(jnp.int32(K))
        zero_hist()

        def p1(i, c):
            na_v, mc_v = c
            u = _mono(rbuf[par, pl.ds(i * 16, 16)])
            d = _digit(u, 27, 5)
            idx = i * 16 + lane
            hi = d > dstar_v
            eq = d == dstar_v
            na_v = append(accu, acci, na_v, u, idx, hi)
            mc_v = append(ua, ia, mc_v, u, idx, eq)
            d2 = _digit(u, 22, 5)
            plsc.addupdate_scatter(hist, [lane, d2], ones_i, mask=eq)
            return na_v, mc_v

        na_v, _ = lax.fori_loop(0, NP // 16, p1, (zero_v, zero_v))
        krem = jnp.int32(K) - csum
        m = jnp.where(krem == 0, 0, cstar)

        def cond(carry):
            p, krem, m, na_v = carry
            return (p < 7) & (krem > 0)

        def pass_body(carry):
            p, krem, m, na_v = carry
            shift = jnp.maximum(27 - 5 * p, 0)
            dmask = jnp.where(p == 6, 3, 31)
            shift2 = jnp.maximum(27 - 5 * (p + 1), 0)
            dmask2 = jnp.where(p + 1 == 6, 3, 31)
            csum, dstar_v, cstar = scan_hist(krem)
            zero_hist()
            trips = (m + 15) // 16

            def pp(i, c):
                na_v, mc_v = c
                lm = (i * 16 + lane) < m
                u = ua[pl.ds(i * 16, 16)]
                idx = ia[pl.ds(i * 16, 16)]
                d = lax.shift_right_logical(u, shift) & dmask
                hi = lm & (d > dstar_v)
                eq = lm & (d == dstar_v)
                na_v = append(accu, acci, na_v, u, idx, hi)
                mc_v = append(ua, ia, mc_v, u, idx, eq)
                d2 = lax.shift_right_logical(u, shift2) & dmask2
                plsc.addupdate_scatter(hist, [lane, d2], ones_i, mask=eq)
                return na_v, mc_v

            na_v, _ = lax.fori_loop(0, trips, pp, (na_v, zero_v))
            krem = krem - csum
            m = jnp.where(krem == 0, 0, cstar)
            return p + 1, krem, m, na_v

        _, krem, m, na_v = lax.while_loop(
            cond, pass_body, (jnp.int32(1), krem, m, na_v))

        def fb(i, na_v):
            lm = (i * 16 + lane) < krem
            u = ua[pl.ds(i * 16, 16)]
            idx = ia[pl.ds(i * 16, 16)]
            return append(accu, acci, na_v, u, idx, lm)

        lax.fori_loop(0, (krem + 15) // 16, fb, na_v)

    def emit_row(row, stage):
        for t in range(K // 16):
            sl = pl.ds(t * 16, 16)
            u = accu[sl]
            idx = acci[sl]
            stage[0, sl] = _unmono(u)
            stage[1, sl] = plsc.load_gather(xx, [idx])
            stage[2, sl] = plsc.load_gather(yy, [idx])
        pltpu.sync_copy(stage, out_hbm.at[row])

    def start_row(row, par, sem):
        return pltpu.async_copy(tr_hbm.at[row], rbuf.at[par], sem)

    start_row(row0, 0, sem_r0)

    def jbody(j, _):
        base = row0 + 2 * j
        pltpu.make_async_copy(tr_hbm.at[base], rbuf.at[0], sem_r0).wait()
        start_row(base + 1, 1, sem_r1)
        select_row(0)
        emit_row(base, stage0)
        pltpu.make_async_copy(tr_hbm.at[base + 1], rbuf.at[1], sem_r1).wait()
        start_row(jnp.minimum(base + 2, NP - 1), 0, sem_r0)
        select_row(1)
        emit_row(base + 1, stage1)
        return 0

    lax.fori_loop(0, ROWS_W // 2, jbody, 0)
    pltpu.make_async_copy(tr_hbm.at[row0], rbuf.at[0], sem_r0).wait()


def _sc_topk(transport, xy2):
    tr = transport[0]
    xyx = xy2[0, :, 0]
    xyy = xy2[0, :, 1]
    mesh = plsc.VectorSubcoreMesh(core_axis_name="c", subcore_axis_name="s")
    f = pl.kernel(
        _sc_topk_body,
        out_type=jax.ShapeDtypeStruct((NP, 3, K), jnp.float32),
        mesh=mesh,
        compiler_params=pltpu.CompilerParams(needs_layout_passes=False),
        scratch_types=[
            pltpu.VMEM((2, NP), jnp.float32),
            pltpu.VMEM((NP,), jnp.int32),
            pltpu.VMEM((NP,), jnp.int32),
            pltpu.VMEM((16, 32), jnp.int32),
            pltpu.VMEM((K + 32,), jnp.int32),
            pltpu.VMEM((K + 32,), jnp.int32),
            pltpu.VMEM((NP,), jnp.float32),
            pltpu.VMEM((NP,), jnp.float32),
            pltpu.VMEM((3, K), jnp.float32),
            pltpu.VMEM((3, K), jnp.float32),
            pltpu.SemaphoreType.DMA,
            pltpu.SemaphoreType.DMA,
            pltpu.SemaphoreType.DMA,
        ],
    )
    return f(tr, xyx, xyy)


def kernel(coords, all_delta_flow, transport, xy2, out_w1, out_b1, out_gn_w,
           out_gn_b, out_prelu, out_w2, out_b2, knn_w1, knn_b1, knn_gn_w,
           knn_gn_b, knn_prelu, knn_w2, knn_b2, num_iters, scale):
    del all_delta_flow, num_iters
    sc_out = _sc_topk(transport, xy2)
    vals = sc_out[:, 0, :]
    xq = sc_out[:, 1, :]
    yq = sc_out[:, 2, :]
    scale_f = jnp.asarray(scale, jnp.float32)
    r012 = [jnp.reshape(scale_f * (2.0 ** i), (1, 1)) for i in range(3)]
    feats, zmax, zmin, s1, s2 = _run_stage2(vals, xq, yq, coords, r012,
                                            knn_w1, knn_b1)
    out = _run_stage3(feats, zmax, zmin, s1, s2, out_w1, out_b1, out_gn_w,
                      out_gn_b, out_prelu, out_w2, out_b2, knn_gn_w,
                      knn_gn_b, knn_prelu, knn_w2, knn_b2)
    return out[None]


# speedup vs baseline: 1.1208x; 1.1208x over previous
import functools

import jax
import jax.numpy as jnp
import numpy as np
from jax import lax
from jax.experimental import pallas as pl
from jax.experimental.pallas import tpu as pltpu
from jax.experimental.pallas import tpu_sc as plsc

_INTERPRET = False

NP = 8192
K = 128
DK = 22
R = 128
NEG = -1e30
POS = 1e30


def _stage2_body(vals_ref, xq_ref, yq_ref, cx_ref, cy_ref,
                 r0_ref, r1_ref, r2_ref,
                 wc_ref, wx_ref, wy_ref, b1_ref,
                 feats_ref, zmax_ref, zmin_ref, s1_ref, s2_ref):
    corr = vals_ref[...]
    dxv = xq_ref[...] - cx_ref[...]
    dyv = yq_ref[...] - cy_ref[...]
    rs = (r0_ref[0, 0], r1_ref[0, 0], r2_ref[0, 0])

    cols = []
    for lvl in range(3):
        r = rs[lvl]
        dv0 = jnp.round(dxv / r)
        dv1 = jnp.round(dyv / r)
        valid = (jnp.abs(dv0) <= 1.0) & (jnp.abs(dv1) <= 1.0)
        cube = (dv0 + 1.0) * 3.0 + (dv1 + 1.0)
        for b in range(9):
            m = valid & (cube == float(b))
            add = jnp.sum(jnp.where(m, corr, 0.0), axis=1)
            cnt = jnp.sum(jnp.where(m, 1.0, 0.0), axis=1)
            cols.append(add / jnp.maximum(cnt, 1.0))
    feats27 = jnp.stack(cols, axis=1)
    feats_ref[...] = jnp.concatenate(
        [feats27, jnp.zeros((R, 5), jnp.float32)], axis=1)

    dist = dxv * dxv + dyv * dyv
    dl = dist[:, :, None]
    dj = dist[:, None, :]
    il = jax.lax.broadcasted_iota(jnp.int32, (K, K), 0)
    ij = jax.lax.broadcasted_iota(jnp.int32, (K, K), 1)
    before = (dl < dj) | ((dl == dj) & (il < ij)[None])
    rank = jnp.sum(before.astype(jnp.float32), axis=1)
    sel = (rank < float(DK))[None]

    wc = wc_ref[...][:, :, None]
    wx = wx_ref[...][:, :, None]
    wy = wy_ref[...][:, :, None]
    b1 = b1_ref[...][:, :, None]
    z = wc * corr[None] + wx * dxv[None] + wy * dyv[None] + b1
    zmax_ref[...] = jnp.max(jnp.where(sel, z, NEG), axis=2)
    zmin_ref[...] = jnp.min(jnp.where(sel, z, POS), axis=2)
    zs = jnp.where(sel, z, 0.0)
    s1_ref[...] = jnp.sum(zs, axis=2)
    s2_ref[...] = jnp.sum(zs * zs, axis=2)


def _stage3_body(feats_ref, zmax_ref, zmin_ref, s1_ref, s2_ref,
                 w1_ref, b1_ref, gnw_ref, gnb_ref, ap_ref, w2_ref, b2_ref,
                 kgnw_ref, kgnb_ref, kap_ref, kw2_ref, kb2_ref, out_ref):
    ic = jax.lax.broadcasted_iota(jnp.int32, (64, 8), 0)
    ig = jax.lax.broadcasted_iota(jnp.int32, (64, 8), 1)
    oneh = ((ic // 8) == ig).astype(jnp.float32)

    x = feats_ref[...].T
    y1 = jnp.dot(w1_ref[...], x, preferred_element_type=jnp.float32)
    y1 = y1 + b1_ref[...]
    nv = jnp.float32(8 * NP)
    sy = jnp.sum(y1, axis=1, keepdims=True)
    mgv = jnp.dot(oneh.T, sy, preferred_element_type=jnp.float32) / nv
    m64v = jnp.dot(oneh, mgv, preferred_element_type=jnp.float32)
    yc = y1 - m64v
    sv = jnp.sum(yc * yc, axis=1, keepdims=True)
    vgv = jnp.dot(oneh.T, sv, preferred_element_type=jnp.float32) / nv
    invv = jnp.dot(oneh, 1.0 / jnp.sqrt(vgv + 1e-5),
                   preferred_element_type=jnp.float32)
    yn = yc * invv * gnw_ref[...] + gnb_ref[...]
    a = ap_ref[0, 0]
    ya = jnp.where(yn >= 0, yn, a * yn)
    voxel = jnp.dot(w2_ref[...], ya, preferred_element_type=jnp.float32)
    voxel = voxel + b2_ref[...]

    n = jnp.float32(8 * NP * DK)
    s1c = jnp.sum(s1_ref[...], axis=1, keepdims=True)
    s2c = jnp.sum(s2_ref[...], axis=1, keepdims=True)
    mg = jnp.dot(oneh.T, s1c, preferred_element_type=jnp.float32) / n
    eg = jnp.dot(oneh.T, s2c, preferred_element_type=jnp.float32) / n
    vg = jnp.maximum(eg - mg * mg, 0.0)
    m64 = jnp.dot(oneh, mg, preferred_element_type=jnp.float32)
    inv64 = jnp.dot(oneh, 1.0 / jnp.sqrt(vg + 1e-5),
                    preferred_element_type=jnp.float32)
    kgnw = kgnw_ref[...]
    znmax = (zmax_ref[...] - m64) * inv64 * kgnw + kgnb_ref[...]
    znmin = (zmin_ref[...] - m64) * inv64 * kgnw + kgnb_ref[...]
    tsel = jnp.where(kgnw >= 0, znmax, znmin)
    ka = kap_ref[0, 0]
    t = jnp.where(tsel >= 0, tsel, ka * tsel)
    knn = jnp.dot(kw2_ref[...], t, preferred_element_type=jnp.float32)
    knn = knn + kb2_ref[...]
    out_ref[...] = voxel + knn


def _run_stage2(vals, xq, yq, coords, r012, knn_w1, knn_b1):
    grid = NP // R
    row_spec = pl.BlockSpec((R, K), lambda i: (i, 0))
    coord_spec = pl.BlockSpec((R, 1), lambda i: (i, 0))
    scalar_spec = pl.BlockSpec((1, 1), lambda i: (0, 0))
    col_spec = pl.BlockSpec((64, 1), lambda i: (0, 0))
    out_specs = [
        pl.BlockSpec((R, 32), lambda i: (i, 0)),
        pl.BlockSpec((64, R), lambda i: (0, i)),
        pl.BlockSpec((64, R), lambda i: (0, i)),
        pl.BlockSpec((64, R), lambda i: (0, i)),
        pl.BlockSpec((64, R), lambda i: (0, i)),
    ]
    out_shape = [
        jax.ShapeDtypeStruct((NP, 32), jnp.float32),
        jax.ShapeDtypeStruct((64, NP), jnp.float32),
        jax.ShapeDtypeStruct((64, NP), jnp.float32),
        jax.ShapeDtypeStruct((64, NP), jnp.float32),
        jax.ShapeDtypeStruct((64, NP), jnp.float32),
    ]
    wc = knn_w1[:, 0:1]
    wx = knn_w1[:, 1:2]
    wy = knn_w1[:, 2:3]
    return pl.pallas_call(
        _stage2_body,
        grid=(grid,),
        in_specs=[row_spec, row_spec, row_spec, coord_spec, coord_spec,
                  scalar_spec, scalar_spec, scalar_spec,
                  col_spec, col_spec, col_spec, col_spec],
        out_specs=out_specs,
        out_shape=out_shape,
        interpret=_INTERPRET,
    )(vals, xq, yq, coords[0, :, 0:1], coords[0, :, 1:2],
      r012[0], r012[1], r012[2], wc, wx, wy, knn_b1[:, None])


def _run_stage3(feats, zmax, zmin, s1, s2, out_w1, out_b1, out_gn_w,
                out_gn_b, out_prelu, out_w2, out_b2, knn_gn_w, knn_gn_b,
                knn_prelu, knn_w2, knn_b2):
    w1p = jnp.concatenate(
        [out_w1, jnp.zeros((64, 5), jnp.float32)], axis=1)
    return pl.pallas_call(
        _stage3_body,
        out_shape=jax.ShapeDtypeStruct((64, NP), jnp.float32),
        interpret=_INTERPRET,
    )(feats, zmax, zmin, s1, s2,
      w1p, out_b1[:, None], out_gn_w[:, None], out_gn_b[:, None],
      out_prelu[:, None], out_w2, out_b2[:, None],
      knn_gn_w[:, None], knn_gn_b[:, None], knn_prelu[:, None],
      knn_w2, knn_b2[:, None])


NW = 32
ROWS_W = NP // NW
MININT = -2147483648


def _mono(v):
    b = plsc.bitcast(v, jnp.int32)
    m = (b >> 31) | jnp.int32(MININT)
    return b ^ m


def _unmono(u):
    t = u >> 31
    return plsc.bitcast(u ^ ((~t) | jnp.int32(MININT)), jnp.float32)


def _digit(u, shift, bits):
    return lax.shift_right_logical(u, shift) & ((1 << bits) - 1)


def _popcnt(mask):
    return jnp.sum(mask.astype(jnp.int32))


def _sc_topk_body(tr_hbm, xyx_hbm, xyy_hbm, out_hbm,
                  rbuf, ua, ia, hist, accu, acci, xx, yy,
                  stage0, stage1, sem_r0, sem_r1, sem_xy):
    wid = lax.axis_index("s") * 2 + lax.axis_index("c")
    lane = lax.iota(jnp.int32, 16)
    ones_i = jnp.ones((16,), jnp.int32)

    pltpu.async_copy(xyx_hbm, xx, sem_xy).wait()
    pltpu.async_copy(xyy_hbm, yy, sem_xy).wait()

    row0 = wid * ROWS_W

    zero_v = jnp.zeros((16,), jnp.int32)

    def zero_hist():
        for r in range(16):
            hist[r, pl.ds(0, 16)] = zero_v
            hist[r, pl.ds(16, 16)] = zero_v

    def scan_hist(krem):
        tot_lo = hist[0, pl.ds(0, 16)]
        tot_hi = hist[0, pl.ds(16, 16)]
        for r in range(1, 16):
            tot_lo = tot_lo + hist[r, pl.ds(0, 16)]
            tot_hi = tot_hi + hist[r, pl.ds(16, 16)]
        s_hi = jnp.sum(tot_hi)
        cum_hi = plsc.cumsum(lax.rev(tot_hi, (0,)))
        cum_lo = plsc.cumsum(lax.rev(tot_lo, (0,)))
        p_hi = plsc.all_reduce_ffs(cum_hi >= krem)
        p_lo = plsc.all_reduce_ffs(cum_lo >= (krem - s_hi))
        dstar_v = jnp.where(s_hi >= krem, 31 - p_hi, 15 - p_lo)
        dh = lane + 16
        csum = (jnp.sum(jnp.where(dh > dstar_v, tot_hi, 0))
                + jnp.sum(jnp.where(lane > dstar_v, tot_lo, 0)))
        cstar = jnp.sum(jnp.where(dh == dstar_v, tot_hi, 0)
                        + jnp.where(lane == dstar_v, tot_lo, 0))
        dstar = jnp.max(dstar_v)
        return csum, dstar, cstar

    def select_row(par):
        zero_hist()

        def h1(i, _):
            u = _mono(rbuf[par, pl.ds(i * 16, 16)])
            d = _digit(u, 27, 5)
            plsc.addupdate_scatter(hist, [lane, d], ones_i)
            return 0

        lax.fori_loop(0, NP // 16, h1, 0)
        csum, dstar, cstar = scan_hist(jnp.int32(K))

        def p1(i, c):
            na, mc = c
            u = _mono(rbuf[par, pl.ds(i * 16, 16)])
            d = _digit(u, 27, 5)
            idx = i * 16 + lane
            hi = d > dstar
            eq = d == dstar
            plsc.store_compressed(accu.at[pl.ds(na, 16)], u, mask=hi)
            plsc.store_compressed(acci.at[pl.ds(na, 16)], idx, mask=hi)
            plsc.store_compressed(ua.at[pl.ds(mc, 16)], u, mask=eq)
            plsc.store_compressed(ia.at[pl.ds(mc, 16)], idx, mask=eq)
            return na + _popcnt(hi), mc + _popcnt(eq)

        na, m = lax.fori_loop(0, NP // 16, p1, (jnp.int32(0), jnp.int32(0)))
        krem = jnp.int32(K) - csum
        m = jnp.where(krem == 0, 0, m)

        def pass_cond(carry):
            p, na, krem, m = carry
            return (p < 7) & (krem > 0)

        def pass_body(carry):
            p, na, krem, m = carry
            shift = jnp.maximum(27 - 5 * p, 0)
            dmask = jnp.where(p == 6, 3, 31)
            zero_hist()
            trips = (m + 15) // 16

            def hp(i, _):
                lm = (i * 16 + lane) < m
                u = ua[pl.ds(i * 16, 16)]
                d = lax.shift_right_logical(u, shift) & dmask
                plsc.addupdate_scatter(hist, [lane, d], ones_i, mask=lm)
                return 0

            lax.fori_loop(0, trips, hp, 0)
            csum, dstar, cstar = scan_hist(krem)

            def pp(i, c):
                na, mc = c
                lm = (i * 16 + lane) < m
                u = ua[pl.ds(i * 16, 16)]
                idx = ia[pl.ds(i * 16, 16)]
                d = lax.shift_right_logical(u, shift) & dmask
                hi = lm & (d > dstar)
                eq = lm & (d == dstar)
                plsc.store_compressed(accu.at[pl.ds(na, 16)], u, mask=hi)
                plsc.store_compressed(acci.at[pl.ds(na, 16)], idx, mask=hi)
                plsc.store_compressed(ua.at[pl.ds(mc, 16)], u, mask=eq)
                plsc.store_compressed(ia.at[pl.ds(mc, 16)], idx, mask=eq)
                return na + _popcnt(hi), mc + _popcnt(eq)

            na, mnew = lax.fori_loop(0, trips, pp, (na, jnp.int32(0)))
            krem = krem - csum
            m = jnp.where(krem == 0, 0, cstar)
            return p + 1, na, krem, m

        _, na, krem, m = lax.while_loop(
            pass_cond, pass_body, (jnp.int32(1), na, krem, m))

        def fb(i, na_):
            lm = (i * 16 + lane) < krem
            u = ua[pl.ds(i * 16, 16)]
            idx = ia[pl.ds(i * 16, 16)]
            plsc.store_compressed(accu.at[pl.ds(na_, 16)], u, mask=lm)
            plsc.store_compressed(acci.at[pl.ds(na_, 16)], idx, mask=lm)
            return na_ + _popcnt(lm)

        lax.fori_loop(0, (krem + 15) // 16, fb, na)

    def emit_row(row, stage):
        for t in range(K // 16):
            sl = pl.ds(t * 16, 16)
            u = accu[sl]
            idx = acci[sl]
            stage[0, sl] = _unmono(u)
            stage[1, sl] = plsc.load_gather(xx, [idx])
            stage[2, sl] = plsc.load_gather(yy, [idx])
        pltpu.sync_copy(stage, out_hbm.at[row])

    def start_row(row, par, sem):
        return pltpu.async_copy(tr_hbm.at[row], rbuf.at[par], sem)

    start_row(row0, 0, sem_r0)

    def jbody(j, _):
        base = row0 + 2 * j
        pltpu.make_async_copy(tr_hbm.at[base], rbuf.at[0], sem_r0).wait()
        start_row(base + 1, 1, sem_r1)
        select_row(0)
        emit_row(base, stage0)
        pltpu.make_async_copy(tr_hbm.at[base + 1], rbuf.at[1], sem_r1).wait()
        start_row(jnp.minimum(base + 2, NP - 1), 0, sem_r0)
        select_row(1)
        emit_row(base + 1, stage1)
        return 0

    lax.fori_loop(0, ROWS_W // 2, jbody, 0)
    pltpu.make_async_copy(tr_hbm.at[row0], rbuf.at[0], sem_r0).wait()


def _sc_topk(transport, xy2):
    tr = transport[0]
    xyx = xy2[0, :, 0]
    xyy = xy2[0, :, 1]
    mesh = plsc.VectorSubcoreMesh(core_axis_name="c", subcore_axis_name="s")
    f = pl.kernel(
        _sc_topk_body,
        out_type=jax.ShapeDtypeStruct((NP, 3, K), jnp.float32),
        mesh=mesh,
        compiler_params=pltpu.CompilerParams(needs_layout_passes=False),
        scratch_types=[
            pltpu.VMEM((2, NP), jnp.float32),
            pltpu.VMEM((NP,), jnp.int32),
            pltpu.VMEM((NP,), jnp.int32),
            pltpu.VMEM((16, 32), jnp.int32),
            pltpu.VMEM((K + 32,), jnp.int32),
            pltpu.VMEM((K + 32,), jnp.int32),
            pltpu.VMEM((NP,), jnp.float32),
            pltpu.VMEM((NP,), jnp.float32),
            pltpu.VMEM((3, K), jnp.float32),
            pltpu.VMEM((3, K), jnp.float32),
            pltpu.SemaphoreType.DMA,
            pltpu.SemaphoreType.DMA,
            pltpu.SemaphoreType.DMA,
        ],
    )
    return f(tr, xyx, xyy)


def kernel(coords, all_delta_flow, transport, xy2, out_w1, out_b1, out_gn_w,
           out_gn_b, out_prelu, out_w2, out_b2, knn_w1, knn_b1, knn_gn_w,
           knn_gn_b, knn_prelu, knn_w2, knn_b2, num_iters, scale):
    del all_delta_flow, num_iters
    sc_out = _sc_topk(transport, xy2)
    vals = sc_out[:, 0, :]
    xq = sc_out[:, 1, :]
    yq = sc_out[:, 2, :]
    scale_f = jnp.asarray(scale, jnp.float32)
    r012 = [jnp.reshape(scale_f * (2.0 ** i), (1, 1)) for i in range(3)]
    feats, zmax, zmin, s1, s2 = _run_stage2(vals, xq, yq, coords, r012,
                                            knn_w1, knn_b1)
    out = _run_stage3(feats, zmax, zmin, s1, s2, out_w1, out_b1, out_gn_w,
                      out_gn_b, out_prelu, out_w2, out_b2, knn_gn_w,
                      knn_gn_b, knn_prelu, knn_w2, knn_b2)
    return out[None]


# speedup vs baseline: 1.1569x; 1.0322x over previous
import functools

import jax
import jax.numpy as jnp
import numpy as np
from jax import lax
from jax.experimental import pallas as pl
from jax.experimental.pallas import tpu as pltpu
from jax.experimental.pallas import tpu_sc as plsc

_INTERPRET = False

NP = 8192
K = 128
DK = 22
R = 128
NEG = -1e30
POS = 1e30


def _stage2_body(vals_ref, xq_ref, yq_ref, cx_ref, cy_ref,
                 r0_ref, r1_ref, r2_ref,
                 wc_ref, wx_ref, wy_ref, b1_ref,
                 feats_ref, zmax_ref, zmin_ref, s1_ref, s2_ref):
    corr = vals_ref[...]
    dxv = xq_ref[...] - cx_ref[...]
    dyv = yq_ref[...] - cy_ref[...]
    rs = (r0_ref[0, 0], r1_ref[0, 0], r2_ref[0, 0])

    cols = []
    for lvl in range(3):
        r = rs[lvl]
        dv0 = jnp.round(dxv / r)
        dv1 = jnp.round(dyv / r)
        valid = (jnp.abs(dv0) <= 1.0) & (jnp.abs(dv1) <= 1.0)
        cube = (dv0 + 1.0) * 3.0 + (dv1 + 1.0)
        for b in range(9):
            m = valid & (cube == float(b))
            add = jnp.sum(jnp.where(m, corr, 0.0), axis=1)
            cnt = jnp.sum(jnp.where(m, 1.0, 0.0), axis=1)
            cols.append(add / jnp.maximum(cnt, 1.0))
    feats27 = jnp.stack(cols, axis=1)
    feats_ref[...] = jnp.concatenate(
        [feats27, jnp.zeros((R, 5), jnp.float32)], axis=1)

    dist = dxv * dxv + dyv * dyv
    dl = dist[:, :, None]
    dj = dist[:, None, :]
    il = jax.lax.broadcasted_iota(jnp.int32, (K, K), 0)
    ij = jax.lax.broadcasted_iota(jnp.int32, (K, K), 1)
    before = (dl < dj) | ((dl == dj) & (il < ij)[None])
    rank = jnp.sum(before.astype(jnp.float32), axis=1)
    sel = (rank < float(DK))[None]

    wc = wc_ref[...][:, :, None]
    wx = wx_ref[...][:, :, None]
    wy = wy_ref[...][:, :, None]
    b1 = b1_ref[...][:, :, None]
    z = wc * corr[None] + wx * dxv[None] + wy * dyv[None] + b1
    zmax_ref[...] = jnp.max(jnp.where(sel, z, NEG), axis=2)
    zmin_ref[...] = jnp.min(jnp.where(sel, z, POS), axis=2)
    zs = jnp.where(sel, z, 0.0)
    s1_ref[...] = jnp.sum(zs, axis=2)
    s2_ref[...] = jnp.sum(zs * zs, axis=2)


def _stage3_body(feats_ref, zmax_ref, zmin_ref, s1_ref, s2_ref,
                 w1_ref, b1_ref, gnw_ref, gnb_ref, ap_ref, w2_ref, b2_ref,
                 kgnw_ref, kgnb_ref, kap_ref, kw2_ref, kb2_ref, out_ref):
    ic = jax.lax.broadcasted_iota(jnp.int32, (64, 8), 0)
    ig = jax.lax.broadcasted_iota(jnp.int32, (64, 8), 1)
    oneh = ((ic // 8) == ig).astype(jnp.float32)

    x = feats_ref[...].T
    y1 = jnp.dot(w1_ref[...], x, preferred_element_type=jnp.float32)
    y1 = y1 + b1_ref[...]
    nv = jnp.float32(8 * NP)
    sy = jnp.sum(y1, axis=1, keepdims=True)
    mgv = jnp.dot(oneh.T, sy, preferred_element_type=jnp.float32) / nv
    m64v = jnp.dot(oneh, mgv, preferred_element_type=jnp.float32)
    yc = y1 - m64v
    sv = jnp.sum(yc * yc, axis=1, keepdims=True)
    vgv = jnp.dot(oneh.T, sv, preferred_element_type=jnp.float32) / nv
    invv = jnp.dot(oneh, 1.0 / jnp.sqrt(vgv + 1e-5),
                   preferred_element_type=jnp.float32)
    yn = yc * invv * gnw_ref[...] + gnb_ref[...]
    a = ap_ref[0, 0]
    ya = jnp.where(yn >= 0, yn, a * yn)
    voxel = jnp.dot(w2_ref[...], ya, preferred_element_type=jnp.float32)
    voxel = voxel + b2_ref[...]

    n = jnp.float32(8 * NP * DK)
    s1c = jnp.sum(s1_ref[...], axis=1, keepdims=True)
    s2c = jnp.sum(s2_ref[...], axis=1, keepdims=True)
    mg = jnp.dot(oneh.T, s1c, preferred_element_type=jnp.float32) / n
    eg = jnp.dot(oneh.T, s2c, preferred_element_type=jnp.float32) / n
    vg = jnp.maximum(eg - mg * mg, 0.0)
    m64 = jnp.dot(oneh, mg, preferred_element_type=jnp.float32)
    inv64 = jnp.dot(oneh, 1.0 / jnp.sqrt(vg + 1e-5),
                    preferred_element_type=jnp.float32)
    kgnw = kgnw_ref[...]
    znmax = (zmax_ref[...] - m64) * inv64 * kgnw + kgnb_ref[...]
    znmin = (zmin_ref[...] - m64) * inv64 * kgnw + kgnb_ref[...]
    tsel = jnp.where(kgnw >= 0, znmax, znmin)
    ka = kap_ref[0, 0]
    t = jnp.where(tsel >= 0, tsel, ka * tsel)
    knn = jnp.dot(kw2_ref[...], t, preferred_element_type=jnp.float32)
    knn = knn + kb2_ref[...]
    out_ref[...] = voxel + knn


def _run_stage2(vals, xq, yq, coords, r012, knn_w1, knn_b1):
    grid = NP // R
    row_spec = pl.BlockSpec((R, K), lambda i: (i, 0))
    coord_spec = pl.BlockSpec((R, 1), lambda i: (i, 0))
    scalar_spec = pl.BlockSpec((1, 1), lambda i: (0, 0))
    col_spec = pl.BlockSpec((64, 1), lambda i: (0, 0))
    out_specs = [
        pl.BlockSpec((R, 32), lambda i: (i, 0)),
        pl.BlockSpec((64, R), lambda i: (0, i)),
        pl.BlockSpec((64, R), lambda i: (0, i)),
        pl.BlockSpec((64, R), lambda i: (0, i)),
        pl.BlockSpec((64, R), lambda i: (0, i)),
    ]
    out_shape = [
        jax.ShapeDtypeStruct((NP, 32), jnp.float32),
        jax.ShapeDtypeStruct((64, NP), jnp.float32),
        jax.ShapeDtypeStruct((64, NP), jnp.float32),
        jax.ShapeDtypeStruct((64, NP), jnp.float32),
        jax.ShapeDtypeStruct((64, NP), jnp.float32),
    ]
    wc = knn_w1[:, 0:1]
    wx = knn_w1[:, 1:2]
    wy = knn_w1[:, 2:3]
    return pl.pallas_call(
        _stage2_body,
        grid=(grid,),
        in_specs=[row_spec, row_spec, row_spec, coord_spec, coord_spec,
                  scalar_spec, scalar_spec, scalar_spec,
                  col_spec, col_spec, col_spec, col_spec],
        out_specs=out_specs,
        out_shape=out_shape,
        interpret=_INTERPRET,
    )(vals, xq, yq, coords[0, :, 0:1], coords[0, :, 1:2],
      r012[0], r012[1], r012[2], wc, wx, wy, knn_b1[:, None])


def _run_stage3(feats, zmax, zmin, s1, s2, out_w1, out_b1, out_gn_w,
                out_gn_b, out_prelu, out_w2, out_b2, knn_gn_w, knn_gn_b,
                knn_prelu, knn_w2, knn_b2):
    w1p = jnp.concatenate(
        [out_w1, jnp.zeros((64, 5), jnp.float32)], axis=1)
    return pl.pallas_call(
        _stage3_body,
        out_shape=jax.ShapeDtypeStruct((64, NP), jnp.float32),
        interpret=_INTERPRET,
    )(feats, zmax, zmin, s1, s2,
      w1p, out_b1[:, None], out_gn_w[:, None], out_gn_b[:, None],
      out_prelu[:, None], out_w2, out_b2[:, None],
      knn_gn_w[:, None], knn_gn_b[:, None], knn_prelu[:, None],
      knn_w2, knn_b2[:, None])


NW = 32
ROWS_W = NP // NW
MININT = -2147483648


def _mono(v):
    b = plsc.bitcast(v, jnp.int32)
    m = (b >> 31) | jnp.int32(MININT)
    return b ^ m


def _unmono(u):
    t = u >> 31
    return plsc.bitcast(u ^ ((~t) | jnp.int32(MININT)), jnp.float32)


def _digit(u, shift, bits):
    return lax.shift_right_logical(u, shift) & ((1 << bits) - 1)


def _popcnt(mask):
    return jnp.sum(mask.astype(jnp.int32))


def _sc_topk_body(tr_hbm, xyx_hbm, xyy_hbm, out_hbm,
                  rbuf, ua, ia, hist, accu, acci, xx, yy,
                  stage0, stage1, sem_r0, sem_r1, sem_xy):
    wid = lax.axis_index("s") * 2 + lax.axis_index("c")
    lane = lax.iota(jnp.int32, 16)
    ones_i = jnp.ones((16,), jnp.int32)

    pltpu.async_copy(xyx_hbm, xx, sem_xy).wait()
    pltpu.async_copy(xyy_hbm, yy, sem_xy).wait()

    row0 = wid * ROWS_W

    def zero_hist():
        z = jnp.zeros((16,), jnp.int32)
        for d in range(32):
            hist[d, :] = z

    def scan_hist(krem):
        csum = jnp.int32(0)
        dstar = jnp.int32(0)
        cstar = jnp.int32(0)
        found = jnp.int32(0)
        for d in range(31, -1, -1):
            c_d = jnp.sum(hist[d, :])
            new = csum + c_d
            cross = (found == 0) & (new >= krem)
            dstar = jnp.where(cross, jnp.int32(d), dstar)
            cstar = jnp.where(cross, c_d, cstar)
            csum = jnp.where((found == 0) & (~cross), new, csum)
            found = found | cross.astype(jnp.int32)
        return csum, dstar, cstar

    def select_row(par):
        zero_hist()

        def h1(i, _):
            u = _mono(rbuf[par, pl.ds(i * 16, 16)])
            d = _digit(u, 27, 5)
            plsc.addupdate_scatter(hist, [d, lane], ones_i)
            return 0

        lax.fori_loop(0, NP // 16, h1, 0)
        csum, dstar, cstar = scan_hist(jnp.int32(K))

        def p1(i, c):
            na, mc = c
            u = _mono(rbuf[par, pl.ds(i * 16, 16)])
            d = _digit(u, 27, 5)
            idx = i * 16 + lane
            hi = d > dstar
            eq = d == dstar
            plsc.store_compressed(accu.at[pl.ds(na, 16)], u, mask=hi)
            plsc.store_compressed(acci.at[pl.ds(na, 16)], idx, mask=hi)
            plsc.store_compressed(ua.at[pl.ds(mc, 16)], u, mask=eq)
            plsc.store_compressed(ia.at[pl.ds(mc, 16)], idx, mask=eq)
            return na + _popcnt(hi), mc + _popcnt(eq)

        na, m = lax.fori_loop(0, NP // 16, p1, (jnp.int32(0), jnp.int32(0)))
        krem = jnp.int32(K) - csum
        m = jnp.where(krem == 0, 0, m)

        def pass_body(p, carry):
            na, krem, m = carry
            shift = jnp.maximum(27 - 5 * p, 0)
            dmask = jnp.where(p == 6, 3, 31)
            zero_hist()
            trips = (m + 15) // 16

            def hp(i, _):
                lm = (i * 16 + lane) < m
                u = ua[pl.ds(i * 16, 16)]
                d = lax.shift_right_logical(u, shift) & dmask
                plsc.addupdate_scatter(hist, [d, lane], ones_i, mask=lm)
                return 0

            lax.fori_loop(0, trips, hp, 0)
            csum, dstar, cstar = scan_hist(krem)

            def pp(i, c):
                na, mc = c
                lm = (i * 16 + lane) < m
                u = ua[pl.ds(i * 16, 16)]
                idx = ia[pl.ds(i * 16, 16)]
                d = lax.shift_right_logical(u, shift) & dmask
                hi = lm & (d > dstar)
                eq = lm & (d == dstar)
                plsc.store_compressed(accu.at[pl.ds(na, 16)], u, mask=hi)
                plsc.store_compressed(acci.at[pl.ds(na, 16)], idx, mask=hi)
                plsc.store_compressed(ua.at[pl.ds(mc, 16)], u, mask=eq)
                plsc.store_compressed(ia.at[pl.ds(mc, 16)], idx, mask=eq)
                return na + _popcnt(hi), mc + _popcnt(eq)

            na, mnew = lax.fori_loop(0, trips, pp, (na, jnp.int32(0)))
            krem = krem - csum
            m = jnp.where(krem == 0, 0, cstar)
            return na, krem, m

        na, krem, m = lax.fori_loop(1, 7, pass_body, (na, krem, m))

        def fb(i, na_):
            lm = (i * 16 + lane) < krem
            u = ua[pl.ds(i * 16, 16)]
            idx = ia[pl.ds(i * 16, 16)]
            plsc.store_compressed(accu.at[pl.ds(na_, 16)], u, mask=lm)
            plsc.store_compressed(acci.at[pl.ds(na_, 16)], idx, mask=lm)
            return na_ + _popcnt(lm)

        lax.fori_loop(0, (krem + 15) // 16, fb, na)

    def emit_row(row, stage):
        for t in range(K // 16):
            sl = pl.ds(t * 16, 16)
            u = accu[sl]
            idx = acci[sl]
            stage[0, sl] = _unmono(u)
            stage[1, sl] = plsc.load_gather(xx, [idx])
            stage[2, sl] = plsc.load_gather(yy, [idx])
        pltpu.sync_copy(stage, out_hbm.at[row])

    def start_row(row, par, sem):
        return pltpu.async_copy(tr_hbm.at[row], rbuf.at[par], sem)

    start_row(row0, 0, sem_r0)

    def jbody(j, _):
        base = row0 + 2 * j
        pltpu.make_async_copy(tr_hbm.at[base], rbuf.at[0], sem_r0).wait()
        start_row(base + 1, 1, sem_r1)
        select_row(0)
        emit_row(base, stage0)
        pltpu.make_async_copy(tr_hbm.at[base + 1], rbuf.at[1], sem_r1).wait()
        start_row(jnp.minimum(base + 2, NP - 1), 0, sem_r0)
        select_row(1)
        emit_row(base + 1, stage1)
        return 0

    lax.fori_loop(0, ROWS_W // 2, jbody, 0)
    pltpu.make_async_copy(tr_hbm.at[row0], rbuf.at[0], sem_r0).wait()


def _sc_topk(transport, xy2):
    tr = transport[0]
    xyx = xy2[0, :, 0]
    xyy = xy2[0, :, 1]
    mesh = plsc.VectorSubcoreMesh(core_axis_name="c", subcore_axis_name="s")
    f = pl.kernel(
        _sc_topk_body,
        out_type=jax.ShapeDtypeStruct((NP, 3, K), jnp.float32),
        mesh=mesh,
        compiler_params=pltpu.CompilerParams(needs_layout_passes=False),
        scratch_types=[
            pltpu.VMEM((2, NP), jnp.float32),
            pltpu.VMEM((NP,), jnp.int32),
            pltpu.VMEM((NP,), jnp.int32),
            pltpu.VMEM((32, 16), jnp.int32),
            pltpu.VMEM((K + 32,), jnp.int32),
            pltpu.VMEM((K + 32,), jnp.int32),
            pltpu.VMEM((NP,), jnp.float32),
            pltpu.VMEM((NP,), jnp.float32),
            pltpu.VMEM((3, K), jnp.float32),
            pltpu.VMEM((3, K), jnp.float32),
            pltpu.SemaphoreType.DMA,
            pltpu.SemaphoreType.DMA,
            pltpu.SemaphoreType.DMA,
        ],
    )
    return f(tr, xyx, xyy)


def kernel(coords, all_delta_flow, transport, xy2, out_w1, out_b1, out_gn_w,
           out_gn_b, out_prelu, out_w2, out_b2, knn_w1, knn_b1, knn_gn_w,
           knn_gn_b, knn_prelu, knn_w2, knn_b2, num_iters, scale):
    del all_delta_flow, num_iters
    sc_out = _sc_topk(transport, xy2)
    vals = sc_out[:, 0, :]
    xq = sc_out[:, 1, :]
    yq = sc_out[:, 2, :]
    scale_f = jnp.asarray(scale, jnp.float32)
    r012 = [jnp.reshape(scale_f * (2.0 ** i), (1, 1)) for i in range(3)]
    feats, zmax, zmin, s1, s2 = _run_stage2(vals, xq, yq, coords, r012,
                                            knn_w1, knn_b1)
    out = _run_stage3(feats, zmax, zmin, s1, s2, out_w1, out_b1, out_gn_w,
                      out_gn_b, out_prelu, out_w2, out_b2, knn_gn_w,
                      knn_gn_b, knn_prelu, knn_w2, knn_b2)
    return out[None]
